# baseline (speedup 1.0000x reference)
import functools

import jax
import jax.numpy as jnp
from jax import lax
from jax.experimental import pallas as pl
from jax.experimental.pallas import tpu as pltpu
from jax.experimental.pallas import tpu_sc as plsc

V = 1000000
D = 64
F = 32
B = 1024
LU = 50
K = 20
LN = 20

NC = 2
NS = 16
NW = NC * NS
L = 16

CH = 80

UB_W = B // NW
NB_W = (B * K) // NW
U_ROWS_W = UB_W * LU
N_ROWS_W = NB_W * LN
U_CHUNKS = U_ROWS_W // CH
N_CHUNKS = N_ROWS_W // CH
NBAGS_CH = CH // LN
NQ = 8
UB_Q = UB_W // NQ
Q_ROWS = N_ROWS_W // NQ
SEG_CH = 8
SEG_BAGS = SEG_CH * NBAGS_CH

KP = 24
MP = 128
TPB = 12800
TG = 79
TAIL0 = (V // TPB) * TPB
TAIL_A = 1536
TAIL1 = TAIL0 + TAIL_A
TV = TG * TPB

_mesh = plsc.VectorSubcoreMesh(
    core_axis_name="c", subcore_axis_name="s", num_cores=NC, num_subcores=NS
)

_GATHER_DN = lax.GatherDimensionNumbers(
    offset_dims=(), collapsed_slice_dims=(0,), start_index_map=(0,)
)


def _splat(wvec, t):
    return lax.gather(
        wvec,
        jnp.full((L, 1), t, jnp.int32),
        _GATHER_DN,
        (1,),
        mode=lax.GatherScatterMode.PROMISE_IN_BOUNDS,
    )


def _full(x):
    return jnp.full((L,), x, jnp.int32)


@functools.partial(
    pl.kernel,
    out_type=(
        jax.ShapeDtypeStruct((B, D), jnp.float32),
        jax.ShapeDtypeStruct((B * K, D), jnp.float32),
    ),
    mesh=_mesh,
    scratch_types=[
        pltpu.VMEM((UB_W, MP), jnp.float32),
        pltpu.VMEM((UB_W, MP), jnp.float32),
        pltpu.VMEM((UB_Q, KP, MP), jnp.float32),
        pltpu.VMEM((U_ROWS_W,), jnp.int32),
        pltpu.VMEM((U_ROWS_W,), jnp.float32),
        pltpu.VMEM((N_ROWS_W,), jnp.int32),
        pltpu.VMEM((N_ROWS_W,), jnp.float32),
        pltpu.VMEM((CH, 2 * D), jnp.float32),
        pltpu.VMEM((CH, 2 * D), jnp.float32),
        pltpu.VMEM((UB_W, D), jnp.float32),
        pltpu.VMEM((SEG_BAGS, D), jnp.float32),
        pltpu.SemaphoreType.DMA,
        pltpu.SemaphoreType.DMA,
    ],
    compiler_params=pltpu.CompilerParams(
        use_tc_tiling_on_sc=False, needs_layout_passes=False
    ),
)
def _sc_bag_sums(ui, uw, ni, nw, table, out_u, out_n,
                 uiv, uwv, big, uif, uwe, nif, nwe, rb0, rb1,
                 ou_v, on_v, sem0, sem1):
    wid = lax.axis_index("s") * NC + lax.axis_index("c")
    iota = lax.iota(jnp.int32, L)
    zero = jnp.zeros((L,), jnp.float32)

    pltpu.sync_copy(ui.at[pl.ds(wid * UB_W, UB_W)], uiv)
    pltpu.sync_copy(uw.at[pl.ds(wid * UB_W, UB_W)], uwv)

    def rep_u(i, _):
        f = i * L + iota
        b = lax.div(f, _full(LU))
        t = f - b * LU
        o = pl.ds(pl.multiple_of(i * L, L), L)
        uif[o] = plsc.bitcast(plsc.load_gather(uiv, [b, t]), jnp.int32)
        uwe[o] = plsc.load_gather(uwv, [b, t])
        return 0

    lax.fori_loop(0, U_ROWS_W // L, rep_u, 0)

    for q in range(NQ):
        def coords(i):
            f = i * L + iota
            b = lax.div(f, _full(K * LN))
            r = f - b * (K * LN)
            k = lax.div(r, _full(LN))
            t = r - k * LN
            return b, k, t

        pltpu.sync_copy(ni.at[pl.ds(wid * UB_W + q * UB_Q, UB_Q)], big)

        def rep_ni(i, _):
            b, k, t = coords(i)
            o = pl.ds(pl.multiple_of(q * Q_ROWS + i * L, L), L)
            nif[o] = plsc.bitcast(plsc.load_gather(big, [b, k, t]), jnp.int32)
            return 0

        lax.fori_loop(0, Q_ROWS // L, rep_ni, 0)

        pltpu.sync_copy(nw.at[pl.ds(wid * UB_W + q * UB_Q, UB_Q)], big)

        def rep_nw(i, _):
            b, k, t = coords(i)
            o = pl.ds(pl.multiple_of(q * Q_ROWS + i * L, L), L)
            nwe[o] = plsc.load_gather(big, [b, k, t])
            return 0

        lax.fori_loop(0, Q_ROWS // L, rep_nw, 0)

    def zbody(i, _):
        for cc in range(D // L):
            ou_v[i, pl.ds(cc * L, L)] = zero
        return 0

    lax.fori_loop(0, UB_W, zbody, 0)

    def gstart(idxf, c, rb, sem):
        pltpu.async_copy(
            table.at[idxf.at[pl.ds(pl.multiple_of(c * CH, CH), CH)]], rb, sem
        )

    def gwait(idxf, c, rb, sem):
        pltpu.make_async_copy(
            table.at[idxf.at[pl.ds(pl.multiple_of(c * CH, CH), CH)]], rb, sem
        ).wait()

    def wvecs(wf, c):
        return [
            wf[pl.ds(pl.multiple_of(c * CH + g * L, L), L)]
            for g in range(CH // L)
        ]

    def ucompute(c, rb):
        we = wvecs(uwe, c)
        for r in range(CH):
            e = _splat(we[r // L], r % L)
            bag = lax.div(c * CH + r, LU)
            for cc in range(D // L):
                plsc.addupdate(
                    ou_v.at[bag, pl.ds(cc * L, L)],
                    e * rb[r, pl.ds(cc * L, L)],
                )

    def ncompute(c, rb):
        we = wvecs(nwe, c)
        for jj in range(NBAGS_CH):
            acc = [zero] * (D // L)
            for t in range(LN):
                r = jj * LN + t
                e = _splat(we[r // L], r % L)
                for cc in range(D // L):
                    acc[cc] = acc[cc] + e * rb[r, pl.ds(cc * L, L)]
            jloc = lax.rem(c, SEG_CH) * NBAGS_CH + jj
            for cc in range(D // L):
                on_v[jloc, pl.ds(cc * L, L)] = acc[cc]

    def run_phase(idxf, nch, compute, flush):
        gstart(idxf, 0, rb0, sem0)

        def body(c2, _):
            c = c2 * 2
            gstart(idxf, c + 1, rb1, sem1)
            gwait(idxf, c, rb0, sem0)
            compute(c, rb0)

            @pl.when(c + 2 < nch)
            def _():
                gstart(idxf, c + 2, rb0, sem0)

            gwait(idxf, c + 1, rb1, sem1)
            compute(c + 1, rb1)
            if flush is not None:
                @pl.when(lax.rem(c2, SEG_CH // 2) == SEG_CH // 2 - 1)
                def _():
                    flush(lax.div(c2, SEG_CH // 2))
            return 0

        lax.fori_loop(0, nch // 2, body, 0)

    def nflush(seg):
        pltpu.sync_copy(
            on_v, out_n.at[pl.ds(wid * NB_W + seg * SEG_BAGS, SEG_BAGS)]
        )

    run_phase(uif, U_CHUNKS, ucompute, None)
    run_phase(nif, N_CHUNKS, ncompute, nflush)

    pltpu.sync_copy(ou_v, out_u.at[pl.ds(wid * UB_W, UB_W)])


def _tf_body(tt_ref, tail_ref, w3t_ref, out_ref, va, sem):
    g = pl.program_id(0)
    w3t = w3t_ref[...]

    @pl.when(g < TG - 1)
    def _():
        cp = pltpu.make_async_copy(tt_ref.at[:, pl.ds(g * TPB, TPB)], va, sem)
        cp.start()
        cp.wait()

    @pl.when(g == TG - 1)
    def _():
        cp = pltpu.make_async_copy(
            tt_ref.at[:, pl.ds(TAIL0, TAIL_A)], va.at[:, pl.ds(0, TAIL_A)], sem
        )
        cp.start()
        cp.wait()

    y = lax.dot_general(
        va[...], w3t, (((0,), (0,)), ((), ())),
        preferred_element_type=jnp.float32,
    )
    out_ref[...] = jnp.concatenate([y, y], axis=1)

    @pl.when(g == TG - 1)
    def _():
        yt = lax.dot_general(
            tail_ref[...], w3t, (((0,), (0,)), ((), ())),
            preferred_element_type=jnp.float32,
        )
        out_ref[pl.ds(TAIL_A, D), :] = jnp.concatenate([yt, yt], axis=1)


def _transform_table(tt, tail, w3t):
    return pl.pallas_call(
        _tf_body,
        grid=(TG,),
        in_specs=[
            pl.BlockSpec(memory_space=pltpu.HBM),
            pl.BlockSpec((D, D), lambda g: (0, 0)),
            pl.BlockSpec((D, D), lambda g: (0, 0)),
        ],
        out_specs=pl.BlockSpec((TPB, 2 * D), lambda g: (g, 0)),
        out_shape=jax.ShapeDtypeStruct((TV, 2 * D), jnp.float32),
        scratch_shapes=[
            pltpu.VMEM((D, TPB), jnp.float32),
            pltpu.SemaphoreType.DMA,
        ],
    )(tt, tail, w3t)


def _mlp_body(ue_ref, ne_ref, b3_ref, w4t_ref, b4_ref, out_ref):
    b3 = b3_ref[...]
    w4t = w4t_ref[...]
    b4 = b4_ref[...]
    uy = jnp.tanh(jnp.tanh(ue_ref[...] + b3) @ w4t + b4)
    ny = jnp.tanh(jnp.tanh(ne_ref[...] + b3) @ w4t + b4)
    un = uy * lax.rsqrt(jnp.sum(uy * uy, axis=1, keepdims=True))
    nn = ny * lax.rsqrt(jnp.sum(ny * ny, axis=1, keepdims=True))
    nn3 = nn.reshape(B, K, F)
    out_ref[...] = jnp.sum(un[:, None, :] * nn3, axis=2)


def _mlp(ue, ne, b3, w4t, b4):
    return pl.pallas_call(
        _mlp_body,
        out_shape=jax.ShapeDtypeStruct((B, K), jnp.float32),
    )(ue, ne, b3, w4t, b4)


def kernel(user_indices, user_weights, user_seq_len, news_indices, news_weights,
           news_seq_len, emb_table, W3, b3, W4, b4):
    del user_seq_len, news_seq_len
    ui_p = lax.bitcast_convert_type(
        jnp.pad(user_indices.astype(jnp.int32), ((0, 0), (0, MP - LU))),
        jnp.float32,
    )
    uw_p = jnp.pad(user_weights, ((0, 0), (0, MP - LU)))
    ni_p = lax.bitcast_convert_type(
        jnp.pad(news_indices.astype(jnp.int32), ((0, 0), (0, KP - K), (0, MP - LN))),
        jnp.float32,
    )
    nw_p = jnp.pad(news_weights, ((0, 0), (0, KP - K), (0, MP - LN)))
    tt = jnp.transpose(emb_table)
    table2 = _transform_table(tt, lax.slice(tt, (0, TAIL1), (D, V)), W3.T)
    ue, ne = _sc_bag_sums(ui_p, uw_p, ni_p, nw_p, table2)
    return _mlp(ue, ne, b3.reshape(1, D), W4.T, b4.reshape(1, F))


# speedup vs baseline: 1.4942x; 1.4942x over previous
import functools

import jax
import jax.numpy as jnp
from jax import lax
from jax.experimental import pallas as pl
from jax.experimental.pallas import tpu as pltpu
from jax.experimental.pallas import tpu_sc as plsc

V = 1000000
D = 64
F = 32
B = 1024
LU = 50
K = 20
LN = 20

NC = 2
NS = 16
NW = NC * NS
L = 16

CH = 80
NBUF = 2

UB_W = B // NW
NB_W = (B * K) // NW
U_ROWS_W = UB_W * LU
N_ROWS_W = NB_W * LN
U_CHUNKS = U_ROWS_W // CH
N_CHUNKS = N_ROWS_W // CH
NBAGS_CH = CH // LN
NQ = 8
UB_Q = UB_W // NQ
Q_ROWS = N_ROWS_W // NQ
SEG_CH = 8
SEG_BAGS = SEG_CH * NBAGS_CH

KP = 24
MP = 128

TPB = 25600
TG = 40
TAIL0 = 39 * TPB
TAIL_A = 1536
TAIL1 = TAIL0 + TAIL_A
TV = TG * TPB

_mesh = plsc.VectorSubcoreMesh(
    core_axis_name="c", subcore_axis_name="s", num_cores=NC, num_subcores=NS
)

_GATHER_DN = lax.GatherDimensionNumbers(
    offset_dims=(), collapsed_slice_dims=(0,), start_index_map=(0,)
)


def _splat(wvec, t):
    return lax.gather(
        wvec,
        jnp.full((L, 1), t, jnp.int32),
        _GATHER_DN,
        (1,),
        mode=lax.GatherScatterMode.PROMISE_IN_BOUNDS,
    )


def _full(x):
    return jnp.full((L,), x, jnp.int32)


@functools.partial(
    pl.kernel,
    out_type=(
        jax.ShapeDtypeStruct((B, D), jnp.float32),
        jax.ShapeDtypeStruct((B * K, D), jnp.float32),
    ),
    mesh=_mesh,
    scratch_types=[
        pltpu.VMEM((UB_W, MP), jnp.float32),
        pltpu.VMEM((UB_W, MP), jnp.float32),
        pltpu.VMEM((UB_Q, KP, MP), jnp.float32),
        pltpu.VMEM((U_ROWS_W,), jnp.int32),
        pltpu.VMEM((U_ROWS_W,), jnp.float32),
        pltpu.VMEM((U_ROWS_W,), jnp.float32),
        pltpu.VMEM((N_ROWS_W,), jnp.int32),
        pltpu.VMEM((N_ROWS_W,), jnp.float32),
        pltpu.VMEM((N_ROWS_W,), jnp.float32),
        pltpu.VMEM((CH, 2 * D), jnp.float32),
        pltpu.VMEM((CH, 2 * D), jnp.float32),
        pltpu.VMEM((UB_W, D), jnp.float32),
        pltpu.VMEM((SEG_BAGS, D), jnp.float32),
        pltpu.SemaphoreType.DMA,
        pltpu.SemaphoreType.DMA,
    ],
    compiler_params=pltpu.CompilerParams(
        use_tc_tiling_on_sc=False, needs_layout_passes=False
    ),
)
def _sc_bag_sums(ui, uw, ni, nw, table, out_u, out_n,
                 uiv, uwv, big, uif, uwe, uwo, nif, nwe, nwo,
                 rb0, rb1, ou_v, on_v, sem0, sem1):
    wid = lax.axis_index("s") * NC + lax.axis_index("c")
    iota = lax.iota(jnp.int32, L)
    zero = jnp.zeros((L,), jnp.float32)
    one = _full(1)

    pltpu.sync_copy(ui.at[pl.ds(wid * UB_W, UB_W)], uiv)
    pltpu.sync_copy(uw.at[pl.ds(wid * UB_W, UB_W)], uwv)

    def rep_u(i, _):
        f = i * L + iota
        b = lax.div(f, _full(LU))
        t = f - b * LU
        o = pl.ds(pl.multiple_of(i * L, L), L)
        uif[o] = plsc.bitcast(plsc.load_gather(uiv, [b, t]), jnp.int32)
        uwe[o] = plsc.load_gather(uwv, [b, t])
        return 0

    lax.fori_loop(0, U_ROWS_W // L, rep_u, 0)

    for q in range(NQ):
        def coords(i):
            f = i * L + iota
            b = lax.div(f, _full(K * LN))
            r = f - b * (K * LN)
            k = lax.div(r, _full(LN))
            t = r - k * LN
            return b, k, t

        pltpu.sync_copy(ni.at[pl.ds(wid * UB_W + q * UB_Q, UB_Q)], big)

        def rep_ni(i, _):
            b, k, t = coords(i)
            o = pl.ds(pl.multiple_of(q * Q_ROWS + i * L, L), L)
            nif[o] = plsc.bitcast(plsc.load_gather(big, [b, k, t]), jnp.int32)
            return 0

        lax.fori_loop(0, Q_ROWS // L, rep_ni, 0)

        pltpu.sync_copy(nw.at[pl.ds(wid * UB_W + q * UB_Q, UB_Q)], big)

        def rep_nw(i, _):
            b, k, t = coords(i)
            o = pl.ds(pl.multiple_of(q * Q_ROWS + i * L, L), L)
            nwe[o] = plsc.load_gather(big, [b, k, t])
            return 0

        lax.fori_loop(0, Q_ROWS // L, rep_nw, 0)

    def zbody(i, _):
        for cc in range(D // L):
            ou_v[i, pl.ds(cc * L, L)] = zero
        return 0

    lax.fori_loop(0, UB_W, zbody, 0)

    def gstart(idxf, c, rb, sem):
        pltpu.async_copy(
            table.at[idxf.at[pl.ds(pl.multiple_of(c * CH, CH), CH)]], rb, sem
        )

    def gwait(idxf, c, rb, sem):
        pltpu.make_async_copy(
            table.at[idxf.at[pl.ds(pl.multiple_of(c * CH, CH), CH)]], rb, sem
        ).wait()

    def wvecs(wf, c):
        return [
            wf[pl.ds(pl.multiple_of(c * CH + g * L, L), L)]
            for g in range(CH // L)
        ]

    def ucompute(c, rb):
        we = wvecs(uwe, c)
        for r in range(CH):
            e = _splat(we[r // L], r % L)
            bag = lax.div(c * CH + r, LU)
            for cc in range(D // L):
                plsc.addupdate(
                    ou_v.at[bag, pl.ds(cc * L, L)],
                    e * rb[r, pl.ds(cc * L, L)],
                )

    def ncompute(c, rb):
        we = wvecs(nwe, c)
        for jj in range(NBAGS_CH):
            acc = [zero] * (D // L)
            for t in range(LN):
                r = jj * LN + t
                e = _splat(we[r // L], r % L)
                for cc in range(D // L):
                    acc[cc] = acc[cc] + e * rb[r, pl.ds(cc * L, L)]
            jloc = lax.rem(c, SEG_CH) * NBAGS_CH + jj
            for cc in range(D // L):
                on_v[jloc, pl.ds(cc * L, L)] = acc[cc]

    bufs = [(rb0, sem0), (rb1, sem1)]

    def run_phase(idxf, nch, compute, flush):
        for u in range(NBUF - 1):
            gstart(idxf, u, *bufs[u])

        def body(c4, _):
            for u in range(NBUF):
                c = c4 * NBUF + u
                nxt = c + NBUF - 1

                @pl.when(nxt < nch)
                def _(u=u, nxt=nxt):
                    gstart(idxf, nxt, *bufs[(u + NBUF - 1) % NBUF])

                gwait(idxf, c, *bufs[u])
                compute(c, bufs[u][0])
            if flush is not None:
                @pl.when(lax.rem(c4, SEG_CH // NBUF) == SEG_CH // NBUF - 1)
                def _():
                    flush(lax.div(c4, SEG_CH // NBUF))
            return 0

        lax.fori_loop(0, nch // NBUF, body, 0)

    def nflush(seg):
        pltpu.sync_copy(
            on_v, out_n.at[pl.ds(wid * NB_W + seg * SEG_BAGS, SEG_BAGS)]
        )

    run_phase(uif, U_CHUNKS, ucompute, None)
    run_phase(nif, N_CHUNKS, ncompute, nflush)

    pltpu.sync_copy(ou_v, out_u.at[pl.ds(wid * UB_W, UB_W)])


def _tf_body(tt_ref, tail_ref, w3t_ref, out_ref, va0, va1, sm0, sm1):
    g = pl.program_id(0)
    w3t = w3t_ref[...]

    def start(i, buf, sem):
        @pl.when(i < TG - 1)
        def _():
            pltpu.make_async_copy(
                tt_ref.at[:, pl.ds(i * TPB, TPB)], buf, sem
            ).start()

        @pl.when(i == TG - 1)
        def _():
            pltpu.make_async_copy(
                tt_ref.at[:, pl.ds(TAIL0, TAIL_A)], buf.at[:, pl.ds(0, TAIL_A)], sem
            ).start()

    def wait(i, buf, sem):
        @pl.when(i < TG - 1)
        def _():
            pltpu.make_async_copy(
                tt_ref.at[:, pl.ds(i * TPB, TPB)], buf, sem
            ).wait()

        @pl.when(i == TG - 1)
        def _():
            pltpu.make_async_copy(
                tt_ref.at[:, pl.ds(TAIL0, TAIL_A)], buf.at[:, pl.ds(0, TAIL_A)], sem
            ).wait()

    @pl.when(g == 0)
    def _():
        start(0, va0, sm0)

    @pl.when(g + 1 < TG)
    def _():
        @pl.when(lax.rem(g, 2) == 0)
        def _():
            start(g + 1, va1, sm1)

        @pl.when(lax.rem(g, 2) == 1)
        def _():
            start(g + 1, va0, sm0)

    def emit(buf, sem):
        wait(g, buf, sem)
        y = lax.dot_general(
            buf[...], w3t, (((0,), (0,)), ((), ())),
            preferred_element_type=jnp.float32,
        )
        out_ref[...] = jnp.concatenate([y, y], axis=1)

    @pl.when(lax.rem(g, 2) == 0)
    def _():
        emit(va0, sm0)

    @pl.when(lax.rem(g, 2) == 1)
    def _():
        emit(va1, sm1)

    @pl.when(g == TG - 1)
    def _():
        yt = lax.dot_general(
            tail_ref[...], w3t, (((0,), (0,)), ((), ())),
            preferred_element_type=jnp.float32,
        )
        out_ref[pl.ds(TAIL_A, D), :] = jnp.concatenate([yt, yt], axis=1)


def _transform_table(tt, tail, w3t):
    return pl.pallas_call(
        _tf_body,
        grid=(TG,),
        in_specs=[
            pl.BlockSpec(memory_space=pltpu.HBM),
            pl.BlockSpec((D, D), lambda g: (0, 0)),
            pl.BlockSpec((D, D), lambda g: (0, 0)),
        ],
        out_specs=pl.BlockSpec((TPB, 2 * D), lambda g: (g, 0)),
        out_shape=jax.ShapeDtypeStruct((TV, 2 * D), jnp.float32),
        scratch_shapes=[
            pltpu.VMEM((D, TPB), jnp.float32),
            pltpu.VMEM((D, TPB), jnp.float32),
            pltpu.SemaphoreType.DMA,
            pltpu.SemaphoreType.DMA,
        ],
    )(tt, tail, w3t)


def _mlp_body(ue_ref, ne_ref, b3_ref, w4t_ref, b4_ref, out_ref):
    b3 = b3_ref[...]
    w4t = w4t_ref[...]
    b4 = b4_ref[...]
    uy = jnp.tanh(jnp.tanh(ue_ref[...] + b3) @ w4t + b4)
    ny = jnp.tanh(jnp.tanh(ne_ref[...] + b3) @ w4t + b4)
    un = uy * lax.rsqrt(jnp.sum(uy * uy, axis=1, keepdims=True))
    nn = ny * lax.rsqrt(jnp.sum(ny * ny, axis=1, keepdims=True))
    nn3 = nn.reshape(B, K, F)
    out_ref[...] = jnp.sum(un[:, None, :] * nn3, axis=2)


def _mlp(ue, ne, b3, w4t, b4):
    return pl.pallas_call(
        _mlp_body,
        out_shape=jax.ShapeDtypeStruct((B, K), jnp.float32),
    )(ue, ne, b3, w4t, b4)


def kernel(user_indices, user_weights, user_seq_len, news_indices, news_weights,
           news_seq_len, emb_table, W3, b3, W4, b4):
    del user_seq_len, news_seq_len
    ui_p = lax.bitcast_convert_type(
        jnp.pad(user_indices.astype(jnp.int32), ((0, 0), (0, MP - LU))),
        jnp.float32,
    )
    uw_p = jnp.pad(user_weights, ((0, 0), (0, MP - LU)))
    ni_p = lax.bitcast_convert_type(
        jnp.pad(news_indices.astype(jnp.int32), ((0, 0), (0, KP - K), (0, MP - LN))),
        jnp.float32,
    )
    nw_p = jnp.pad(news_weights, ((0, 0), (0, KP - K), (0, MP - LN)))
    tt = jnp.transpose(emb_table)
    table2 = _transform_table(tt, lax.slice(tt, (0, TAIL1), (D, V)), W3.T)
    ue, ne = _sc_bag_sums(ui_p, uw_p, ni_p, nw_p, table2)
    return _mlp(ue, ne, b3.reshape(1, D), W4.T, b4.reshape(1, F))
